# baseline (speedup 1.0000x reference)
import jax
import jax.numpy as jnp
from jax import lax
from jax.experimental import pallas as pl
from jax.experimental.pallas import tpu as pltpu
from jax.experimental.pallas import tpu_sc as plsc

B = 16384
N_SPARSE = 26
N_DENSE = 13
VOCAB = 100000
LANES = 16

_info = plsc.get_sparse_core_info()
NC, NS = _info.num_cores, _info.num_subcores
NW = NC * NS
ROWS_PER_W = B // NW
CHUNKS = ROWS_PER_W // LANES


def _sc_body(xs_hbm, xd_hbm, tab_hbm, w_hbm, out_hbm,
             xs_v, xd_v, w_v, idx_v, gat_v, out_v, sem):
    wid = lax.axis_index("s") * NC + lax.axis_index("c")
    base = wid * ROWS_PER_W

    pltpu.sync_copy(xs_hbm.at[pl.ds(base * N_SPARSE, ROWS_PER_W * N_SPARSE)],
                    xs_v)
    pltpu.sync_copy(xd_hbm.at[pl.ds(base * N_DENSE, ROWS_PER_W * N_DENSE)],
                    xd_v)
    pltpu.sync_copy(w_hbm, w_v)

    iota = lax.iota(jnp.int32, LANES)

    def build(c, carry):
        rows = iota + c * LANES
        for f in range(N_SPARSE):
            v = plsc.load_gather(xs_v, [rows * N_SPARSE + f])
            idx_v[pl.ds((c * N_SPARSE + f) * LANES, LANES)] = v + f * VOCAB
        return carry

    lax.fori_loop(0, CHUNKS, build, 0)

    pltpu.async_copy(tab_hbm.at[idx_v], gat_v, sem).wait()

    w_vec = w_v[:]
    w_scal = [w_vec[k] for k in range(N_DENSE)]

    def reduce(c, carry):
        acc = gat_v[pl.ds((c * N_SPARSE) * LANES, LANES)]
        for f in range(1, N_SPARSE):
            acc = acc + gat_v[pl.ds((c * N_SPARSE + f) * LANES, LANES)]
        rows = iota + c * LANES
        for k in range(N_DENSE):
            xdk = plsc.load_gather(xd_v, [rows * N_DENSE + k])
            acc = acc + xdk * w_scal[k]
        out_v[c, :] = acc
        return carry

    lax.fori_loop(0, CHUNKS, reduce, 0)

    pltpu.sync_copy(out_v, out_hbm.at[pl.ds(wid * CHUNKS, CHUNKS)])


@jax.jit
def kernel(X_sparse, X_dense, tables, weight):
    tab_flat = tables.reshape(-1)
    w_pad = jnp.zeros((LANES,), jnp.float32).at[:N_DENSE].set(weight[:, 0])

    mesh = plsc.VectorSubcoreMesh(core_axis_name="c", subcore_axis_name="s")
    run = pl.kernel(
        _sc_body,
        mesh=mesh,
        out_type=jax.ShapeDtypeStruct((B // LANES, LANES), jnp.float32),
        scratch_types=[
            pltpu.VMEM((ROWS_PER_W * N_SPARSE,), jnp.int32),
            pltpu.VMEM((ROWS_PER_W * N_DENSE,), jnp.float32),
            pltpu.VMEM((LANES,), jnp.float32),
            pltpu.VMEM((CHUNKS * N_SPARSE * LANES,), jnp.int32),
            pltpu.VMEM((CHUNKS * N_SPARSE * LANES,), jnp.float32),
            pltpu.VMEM((CHUNKS, LANES), jnp.float32),
            pltpu.SemaphoreType.DMA,
        ],
        compiler_params=pltpu.CompilerParams(needs_layout_passes=False),
    )
    out = run(X_sparse.reshape(-1), X_dense.reshape(-1), tab_flat, w_pad)
    return out.reshape(B, 1)
